# Initial kernel scaffold; baseline (speedup 1.0000x reference)
import jax
import jax.numpy as jnp
from jax.experimental import pallas as pl


def kernel(x_src, x_dst, edge_attr, W_msg, b_msg, W_apply, b_apply, W_edge, b_edge, edge_index):
    raise NotImplementedError("write your pallas kernel here")


# speedup vs baseline: 1.9385x; 1.9385x over previous
import functools

import jax
import jax.numpy as jnp
from jax import lax
from jax.experimental import pallas as pl
from jax.experimental.pallas import tpu as pltpu
from jax.experimental.pallas import tpu_sc as plsc

N = 10000
E = 320000
D = 128
DE = 16

NC = 2
NS = 16
NW = NC * NS
EPW = E // NW
EPS = E // NS
C = 80
NCHUNK = EPW // C
NCHUNK_CNT = EPS // C
OWN = 640
ZR = 80
mesh = plsc.VectorSubcoreMesh(core_axis_name="c", subcore_axis_name="s")


def _tc_node_body(x_ref, w_ref, q1_ref, p1_ref):
    xw = jnp.dot(x_ref[:], w_ref[:], preferred_element_type=jnp.float32)
    q1_ref[:] = xw[:, :D]
    p1_ref[:] = xw[:, D:]


def _tc_node(x_src, wn_t):
    return pl.pallas_call(
        _tc_node_body,
        out_shape=[
            jax.ShapeDtypeStruct((N, D), jnp.float32),
            jax.ShapeDtypeStruct((N, DE), jnp.float32),
        ],
    )(x_src, wn_t)


def _tc_edge_body(attr_ref, w_ref, b_ref, a_ref, a2_ref):
    aw = jnp.dot(attr_ref[:], w_ref[:], preferred_element_type=jnp.float32)
    aw = aw + b_ref[:]
    a_ref[:] = aw[:, :D]
    a2_ref[:] = aw[:, D:]


def _tc_edge(edge_attr, we_t, bias):
    BE = 16000
    grid = E // BE
    return pl.pallas_call(
        _tc_edge_body,
        grid=(grid,),
        in_specs=[
            pl.BlockSpec((BE, DE), lambda i: (i, 0)),
            pl.BlockSpec((DE, D + DE), lambda i: (0, 0)),
            pl.BlockSpec((1, D + DE), lambda i: (0, 0)),
        ],
        out_specs=[
            pl.BlockSpec((BE, D), lambda i: (i, 0)),
            pl.BlockSpec((BE, DE), lambda i: (i, 0)),
        ],
        out_shape=[
            jax.ShapeDtypeStruct((E, D), jnp.float32),
            jax.ShapeDtypeStruct((E, DE), jnp.float32),
        ],
    )(edge_attr, we_t, bias)


@functools.partial(
    pl.kernel,
    out_type=jax.ShapeDtypeStruct((NC, N, D), jnp.float32),
    mesh=mesh,
    scratch_types=[
        pltpu.VMEM((C,), jnp.int32),
        pltpu.VMEM((C,), jnp.int32),
        pltpu.VMEM((C, D), jnp.float32),
        pltpu.VMEM((C, D), jnp.float32),
        pltpu.VMEM((C, D), jnp.float32),
        pltpu.VMEM((ZR, D), jnp.float32),
        pltpu.VMEM_SHARED((N, D), jnp.float32),
        pltpu.SemaphoreType.DMA,
    ],
    compiler_params=pltpu.CompilerParams(use_tc_tiling_on_sc=False),
)
def _sc_scatter(q1_hbm, a_hbm, ei_hbm, out_hbm,
                sidx, didx, qrows, arows, mout, zbuf, acc, sem):
    cid = lax.axis_index("c")
    sid = lax.axis_index("s")
    wid = sid * NC + cid
    row0 = sid * OWN
    nown = jnp.where(sid == NS - 1, N - (NS - 1) * OWN, OWN)
    nchunks_own = nown // ZR
    zv = jnp.zeros((16,), jnp.float32)

    def zrow(e, _):
        for j in range(D // 16):
            zbuf[e, pl.ds(j * 16, 16)] = zv
        return 0

    lax.fori_loop(0, ZR, zrow, 0)

    def zacc(k, _):
        @pl.when(k < nchunks_own)
        def _do():
            pltpu.sync_copy(zbuf, acc.at[pl.ds(row0 + k * ZR, ZR), :])
        return 0

    lax.fori_loop(0, 8, zacc, 0)
    plsc.subcore_barrier()

    ebase = wid * EPW

    def chunk(ci, _):
        base = ebase + ci * C
        pltpu.sync_copy(ei_hbm.at[0, pl.ds(base, C)], sidx)
        pltpu.sync_copy(ei_hbm.at[1, pl.ds(base, C)], didx)
        pltpu.async_copy(q1_hbm.at[sidx], qrows, sem).wait()
        pltpu.sync_copy(a_hbm.at[pl.ds(base, C), :], arows)

        def body(e, _):
            for j in range(D // 16):
                sl = pl.ds(j * 16, 16)
                mout[e, sl] = jnp.maximum(qrows[e, sl] + arows[e, sl], 0.0)
            return 0

        lax.fori_loop(0, C, body, 0)
        pltpu.sync_copy(mout, acc.at[didx], add=True)
        return 0

    lax.fori_loop(0, NCHUNK, chunk, 0)
    plsc.subcore_barrier()

    def cout(k, _):
        @pl.when(k < nchunks_own)
        def _do():
            r0 = row0 + k * ZR
            pltpu.sync_copy(acc.at[pl.ds(r0, ZR), :],
                            out_hbm.at[cid, pl.ds(r0, ZR), :])
        return 0

    lax.fori_loop(0, 8, cout, 0)


@functools.partial(
    pl.kernel,
    out_type=[
        jax.ShapeDtypeStruct((E, DE), jnp.float32),
        jax.ShapeDtypeStruct((NC, N, DE), jnp.float32),
    ],
    mesh=mesh,
    scratch_types=[
        pltpu.VMEM((C,), jnp.int32),
        pltpu.VMEM((C,), jnp.int32),
        pltpu.VMEM((C, DE), jnp.float32),
        pltpu.VMEM((C, DE), jnp.float32),
        pltpu.VMEM((C, DE), jnp.float32),
        pltpu.VMEM((ZR, DE), jnp.float32),
        pltpu.VMEM_SHARED((N, DE), jnp.float32),
        pltpu.SemaphoreType.DMA,
    ],
    compiler_params=pltpu.CompilerParams(use_tc_tiling_on_sc=False),
)
def _sc_count_b(p1_hbm, a2_hbm, ei_hbm, b_hbm, cnt_hbm,
                sidx, didx, p1r, a2r, onesr, zbuf, cnt, sem):
    cid = lax.axis_index("c")
    sid = lax.axis_index("s")
    wid = sid * NC + cid
    row0 = sid * OWN
    nown = jnp.where(sid == NS - 1, N - (NS - 1) * OWN, OWN)
    nchunks_own = nown // ZR
    zv = jnp.zeros((16,), jnp.float32)
    onev = jnp.where(jnp.arange(16, dtype=jnp.int32) == 0,
                     jnp.float32(1.0), jnp.float32(0.0))

    def zrow(e, _):
        zbuf[e, :] = zv
        return 0

    lax.fori_loop(0, ZR, zrow, 0)

    def orow(e, _):
        onesr[e, :] = onev
        return 0

    lax.fori_loop(0, C, orow, 0)

    def zacc(k, _):
        @pl.when(k < nchunks_own)
        def _do():
            pltpu.sync_copy(zbuf, cnt.at[pl.ds(row0 + k * ZR, ZR), :])
        return 0

    lax.fori_loop(0, 8, zacc, 0)
    plsc.subcore_barrier()

    ebase = wid * EPW

    def chunk(ci, _):
        base = ebase + ci * C
        pltpu.sync_copy(ei_hbm.at[0, pl.ds(base, C)], sidx)
        pltpu.sync_copy(ei_hbm.at[1, pl.ds(base, C)], didx)
        pltpu.async_copy(p1_hbm.at[sidx], p1r, sem).wait()
        pltpu.sync_copy(a2_hbm.at[pl.ds(base, C), :], a2r)

        def body(e, _):
            a2r[e, :] = p1r[e, :] + a2r[e, :]
            return 0

        lax.fori_loop(0, C, body, 0)
        pltpu.sync_copy(a2r, b_hbm.at[pl.ds(base, C), :])
        pltpu.sync_copy(onesr, cnt.at[didx], add=True)
        return 0

    lax.fori_loop(0, NCHUNK, chunk, 0)
    plsc.subcore_barrier()

    def cout(k, _):
        @pl.when(k < nchunks_own)
        def _do():
            r0 = row0 + k * ZR
            pltpu.sync_copy(cnt.at[pl.ds(r0, ZR), :],
                            cnt_hbm.at[cid, pl.ds(r0, ZR), :])
        return 0

    lax.fori_loop(0, 8, cout, 0)


def _tc_apply_body(p_ref, c_ref, xd_ref, wa_t_ref, we3_t_ref, ba_ref,
                   h_ref, p3_ref):
    s = p_ref[0] + p_ref[1]
    cnt = jnp.maximum(c_ref[0, :, :1] + c_ref[1, :, :1], 1.0)
    hn = s / cnt
    h_ref[:] = (jnp.dot(xd_ref[:], wa_t_ref[:D, :],
                        preferred_element_type=jnp.float32)
                + jnp.dot(hn, wa_t_ref[D:, :],
                          preferred_element_type=jnp.float32)
                + ba_ref[:])
    p3_ref[:] = jnp.dot(hn, we3_t_ref[:], preferred_element_type=jnp.float32)


def _tc_apply(partials, cnts, x_dst, wa_t, we3_t, ba):
    return pl.pallas_call(
        _tc_apply_body,
        out_shape=[
            jax.ShapeDtypeStruct((N, D), jnp.float32),
            jax.ShapeDtypeStruct((N, DE), jnp.float32),
        ],
    )(partials, cnts, x_dst, wa_t, we3_t, ba)


@functools.partial(
    pl.kernel,
    out_type=jax.ShapeDtypeStruct((E, DE), jnp.float32),
    mesh=mesh,
    scratch_types=[
        pltpu.VMEM((C,), jnp.int32),
        pltpu.VMEM((C, DE), jnp.float32),
        pltpu.VMEM((C, DE), jnp.float32),
        pltpu.SemaphoreType.DMA,
    ],
    compiler_params=pltpu.CompilerParams(use_tc_tiling_on_sc=False),
)
def _sc_edge_out(b_hbm, p3_hbm, ei_hbm, eh_hbm,
                 didx, p3r, br, sem):
    cid = lax.axis_index("c")
    sid = lax.axis_index("s")
    wid = sid * NC + cid
    ebase = wid * EPW

    def chunk(ci, _):
        base = ebase + ci * C
        pltpu.sync_copy(ei_hbm.at[1, pl.ds(base, C)], didx)
        pltpu.async_copy(p3_hbm.at[didx], p3r, sem).wait()
        pltpu.sync_copy(b_hbm.at[pl.ds(base, C), :], br)

        def body(e, _):
            br[e, :] = p3r[e, :] + br[e, :]
            return 0

        lax.fori_loop(0, C, body, 0)
        pltpu.sync_copy(br, eh_hbm.at[pl.ds(base, C), :])
        return 0

    lax.fori_loop(0, NCHUNK, chunk, 0)


def kernel(x_src, x_dst, edge_attr, W_msg, b_msg, W_apply, b_apply,
           W_edge, b_edge, edge_index):
    wn_t = jnp.concatenate([W_msg[:, :D], W_edge[:, :D]], axis=0).T
    we_t = jnp.concatenate([W_msg[:, D:], W_edge[:, D:D + DE]], axis=0).T
    be = jnp.concatenate([b_msg, b_edge]).reshape(1, D + DE)
    wa_t = W_apply.T
    we3_t = W_edge[:, D + DE:].T
    ba = b_apply.reshape(1, D)

    q1, p1 = _tc_node(x_src, wn_t)
    a, a2 = _tc_edge(edge_attr, we_t, be)
    partials = _sc_scatter(q1, a, edge_index)
    b, cnts = _sc_count_b(p1, a2, edge_index)
    h_out, p3 = _tc_apply(partials, cnts, x_dst, wa_t, we3_t, ba)
    eh = _sc_edge_out(b, p3, edge_index)
    return (h_out, eh)


# speedup vs baseline: 3.4346x; 1.7718x over previous
import functools

import jax
import jax.numpy as jnp
from jax import lax
from jax.experimental import pallas as pl
from jax.experimental.pallas import tpu as pltpu
from jax.experimental.pallas import tpu_sc as plsc

N = 10000
E = 320000
D = 128
DE = 16

NC = 2
NS = 16
NW = NC * NS
EPW = E // NW
C = 100
NCHUNK = EPW // C
NPAIR = NCHUNK // 2
CS = 25
NCHUNKS = EPW // CS
NPAIRS = NCHUNKS // 2
OWN = 640
ZR = 80
ZB = 16
mesh = plsc.VectorSubcoreMesh(core_axis_name="c", subcore_axis_name="s")


def _tc_node_body(x_ref, w_ref, q1_ref, p1_ref):
    xw = jnp.dot(x_ref[:], w_ref[:], preferred_element_type=jnp.float32)
    q1_ref[:] = xw[:, :D]
    p1_ref[:] = xw[:, D:]


def _tc_node(x_src, wn_t):
    return pl.pallas_call(
        _tc_node_body,
        out_shape=[
            jax.ShapeDtypeStruct((N, D), jnp.float32),
            jax.ShapeDtypeStruct((N, DE), jnp.float32),
        ],
    )(x_src, wn_t)


def _tc_edge_body(attr_ref, w_ref, b_ref, a_ref, a2_ref):
    aw = jnp.dot(attr_ref[:], w_ref[:], preferred_element_type=jnp.float32)
    aw = aw + b_ref[:]
    a_ref[:] = aw[:, :D]
    a2_ref[:] = aw[:, D:]


def _tc_edge(edge_attr, we_t, bias):
    BE = 16000
    grid = E // BE
    return pl.pallas_call(
        _tc_edge_body,
        grid=(grid,),
        in_specs=[
            pl.BlockSpec((BE, DE), lambda i: (i, 0)),
            pl.BlockSpec((DE, D + DE), lambda i: (0, 0)),
            pl.BlockSpec((1, D + DE), lambda i: (0, 0)),
        ],
        out_specs=[
            pl.BlockSpec((BE, D), lambda i: (i, 0)),
            pl.BlockSpec((BE, DE), lambda i: (i, 0)),
        ],
        out_shape=[
            jax.ShapeDtypeStruct((E, D), jnp.float32),
            jax.ShapeDtypeStruct((E, DE), jnp.float32),
        ],
    )(edge_attr, we_t, bias)


@functools.partial(
    pl.kernel,
    out_type=jax.ShapeDtypeStruct((NC, N, D), jnp.float32),
    mesh=mesh,
    scratch_types=[
        pltpu.VMEM((NCHUNKS, CS), jnp.int32),
        pltpu.VMEM((NCHUNKS, CS), jnp.int32),
        pltpu.VMEM((CS, D), jnp.float32),
        pltpu.VMEM((CS, D), jnp.float32),
        pltpu.VMEM((CS, D), jnp.float32),
        pltpu.VMEM((CS, D), jnp.float32),
        pltpu.VMEM((CS, D), jnp.float32),
        pltpu.VMEM((CS, D), jnp.float32),
        pltpu.VMEM((ZB, D), jnp.float32),
        pltpu.VMEM_SHARED((N, D), jnp.float32),
        pltpu.SemaphoreType.DMA,
        pltpu.SemaphoreType.DMA,
        pltpu.SemaphoreType.DMA,
        pltpu.SemaphoreType.DMA,
        pltpu.SemaphoreType.DMA,
        pltpu.SemaphoreType.DMA,
    ],
    compiler_params=pltpu.CompilerParams(use_tc_tiling_on_sc=False),
)
def _sc_scatter(q1_hbm, a_hbm, src2_hbm, dst2_hbm, out_hbm,
                sidx, didx, q0, q1b, a0, a1, m0, m1, zbuf, acc,
                gq0, gq1, ga0, ga1, ss0, ss1):
    cid = lax.axis_index("c")
    sid = lax.axis_index("s")
    wid = sid * NC + cid
    row0 = sid * OWN
    nown = jnp.where(sid == NS - 1, N - (NS - 1) * OWN, OWN)
    nchunks_own = nown // ZR
    nzb_own = nown // ZB
    zv = jnp.zeros((16,), jnp.float32)

    def zrow(e, _):
        for j in range(D // 16):
            zbuf[e, pl.ds(j * 16, 16)] = zv
        return 0

    lax.fori_loop(0, ZB, zrow, 0)

    def zacc(k, _):
        @pl.when(k < nzb_own)
        def _do():
            pltpu.sync_copy(zbuf, acc.at[pl.ds(row0 + k * ZB, ZB), :])
        return 0

    lax.fori_loop(0, OWN // ZB, zacc, 0)

    pltpu.sync_copy(src2_hbm.at[pl.ds(wid * NCHUNKS, NCHUNKS), :], sidx)
    pltpu.sync_copy(dst2_hbm.at[pl.ds(wid * NCHUNKS, NCHUNKS), :], didx)
    plsc.subcore_barrier()

    ebase = wid * EPW

    def issue_loads(c, qb, ab, qsem, asem):
        pltpu.async_copy(q1_hbm.at[sidx.at[c]], qb, qsem)
        pltpu.async_copy(a_hbm.at[pl.ds(ebase + c * CS, CS), :], ab, asem)

    def drain_loads(c, qb, ab, qsem, asem):
        pltpu.make_async_copy(q1_hbm.at[sidx.at[c]], qb, qsem).wait()
        pltpu.make_async_copy(a_hbm.at[pl.ds(ebase + c * CS, CS), :],
                              ab, asem).wait()

    def compute(qb, ab, mb):
        def body(e, _):
            for j in range(D // 16):
                sl = pl.ds(j * 16, 16)
                mb[e, sl] = jnp.maximum(qb[e, sl] + ab[e, sl], 0.0)
            return 0

        lax.fori_loop(0, CS, body, 0)

    def issue_scatter(c, mb, sem):
        pltpu.async_copy(mb, acc.at[didx.at[c]], sem, add=True)

    def drain_scatter(c, mb, sem):
        pltpu.make_async_copy(mb, acc.at[didx.at[c]], sem).wait()

    issue_loads(0, q0, a0, gq0, ga0)

    def pair(k, _):
        c0 = 2 * k
        c1 = 2 * k + 1
        issue_loads(c1, q1b, a1, gq1, ga1)
        drain_loads(c0, q0, a0, gq0, ga0)

        @pl.when(k > 0)
        def _w0():
            drain_scatter(c0 - 2, m0, ss0)

        compute(q0, a0, m0)
        issue_scatter(c0, m0, ss0)

        @pl.when(k < NPAIRS - 1)
        def _i0():
            issue_loads(c0 + 2, q0, a0, gq0, ga0)

        drain_loads(c1, q1b, a1, gq1, ga1)

        @pl.when(k > 0)
        def _w1():
            drain_scatter(c1 - 2, m1, ss1)

        compute(q1b, a1, m1)
        issue_scatter(c1, m1, ss1)
        return 0

    lax.fori_loop(0, NPAIRS, pair, 0)
    drain_scatter(NCHUNKS - 2, m0, ss0)
    drain_scatter(NCHUNKS - 1, m1, ss1)
    plsc.subcore_barrier()

    def cout(k, _):
        @pl.when(k < nchunks_own)
        def _do():
            r0 = row0 + k * ZR
            pltpu.sync_copy(acc.at[pl.ds(r0, ZR), :],
                            out_hbm.at[cid, pl.ds(r0, ZR), :])
        return 0

    lax.fori_loop(0, 8, cout, 0)


@functools.partial(
    pl.kernel,
    out_type=[
        jax.ShapeDtypeStruct((E, DE), jnp.float32),
        jax.ShapeDtypeStruct((NC, N, DE), jnp.float32),
    ],
    mesh=mesh,
    scratch_types=[
        pltpu.VMEM((NCHUNK, C), jnp.int32),
        pltpu.VMEM((NCHUNK, C), jnp.int32),
        pltpu.VMEM((C, DE), jnp.float32),
        pltpu.VMEM((C, DE), jnp.float32),
        pltpu.VMEM((C, DE), jnp.float32),
        pltpu.VMEM((C, DE), jnp.float32),
        pltpu.VMEM((C, DE), jnp.float32),
        pltpu.VMEM((C, DE), jnp.float32),
        pltpu.VMEM((C, DE), jnp.float32),
        pltpu.VMEM((ZR, DE), jnp.float32),
        pltpu.VMEM_SHARED((N, DE), jnp.float32),
        pltpu.SemaphoreType.DMA,
        pltpu.SemaphoreType.DMA,
        pltpu.SemaphoreType.DMA,
        pltpu.SemaphoreType.DMA,
        pltpu.SemaphoreType.DMA,
        pltpu.SemaphoreType.DMA,
        pltpu.SemaphoreType.DMA,
        pltpu.SemaphoreType.DMA,
    ],
    compiler_params=pltpu.CompilerParams(use_tc_tiling_on_sc=False),
)
def _sc_count_b(p1_hbm, a2_hbm, src2_hbm, dst2_hbm, b_hbm, cnt_hbm,
                sidx, didx, p0, p1b, a0, a1, b0, b1, onesr, zbuf, cnt,
                gp0, gp1, ga0, ga1, sb0, sb1, sc0, sc1):
    cid = lax.axis_index("c")
    sid = lax.axis_index("s")
    wid = sid * NC + cid
    row0 = sid * OWN
    nown = jnp.where(sid == NS - 1, N - (NS - 1) * OWN, OWN)
    nchunks_own = nown // ZR
    zv = jnp.zeros((16,), jnp.float32)
    onev = jnp.where(jnp.arange(16, dtype=jnp.int32) == 0,
                     jnp.float32(1.0), jnp.float32(0.0))

    def zrow(e, _):
        zbuf[e, :] = zv
        return 0

    lax.fori_loop(0, ZR, zrow, 0)

    def orow(e, _):
        onesr[e, :] = onev
        return 0

    lax.fori_loop(0, C, orow, 0)

    def zacc(k, _):
        @pl.when(k < nchunks_own)
        def _do():
            pltpu.sync_copy(zbuf, cnt.at[pl.ds(row0 + k * ZR, ZR), :])
        return 0

    lax.fori_loop(0, 8, zacc, 0)
    pltpu.sync_copy(src2_hbm.at[pl.ds(wid * NCHUNK, NCHUNK), :], sidx)
    pltpu.sync_copy(dst2_hbm.at[pl.ds(wid * NCHUNK, NCHUNK), :], didx)
    plsc.subcore_barrier()

    ebase = wid * EPW

    def issue_loads(c, pb, ab, psem, asem):
        pltpu.async_copy(p1_hbm.at[sidx.at[c]], pb, psem)
        pltpu.async_copy(a2_hbm.at[pl.ds(ebase + c * C, C), :], ab, asem)

    def drain_loads(c, pb, ab, psem, asem):
        pltpu.make_async_copy(p1_hbm.at[sidx.at[c]], pb, psem).wait()
        pltpu.make_async_copy(a2_hbm.at[pl.ds(ebase + c * C, C), :],
                              ab, asem).wait()

    def compute(pb, ab, bb):
        def body(e, _):
            bb[e, :] = pb[e, :] + ab[e, :]
            return 0

        lax.fori_loop(0, C, body, 0)

    def issue_out(c, bb, bsem, csem):
        pltpu.async_copy(bb, b_hbm.at[pl.ds(ebase + c * C, C), :], bsem)
        pltpu.async_copy(onesr, cnt.at[didx.at[c]], csem, add=True)

    def drain_out(c, bb, bsem, csem):
        pltpu.make_async_copy(bb, b_hbm.at[pl.ds(ebase + c * C, C), :],
                              bsem).wait()
        pltpu.make_async_copy(onesr, cnt.at[didx.at[c]], csem).wait()

    issue_loads(0, p0, a0, gp0, ga0)

    def pair(k, _):
        c0 = 2 * k
        c1 = 2 * k + 1
        issue_loads(c1, p1b, a1, gp1, ga1)
        drain_loads(c0, p0, a0, gp0, ga0)

        @pl.when(k > 0)
        def _w0():
            drain_out(c0 - 2, b0, sb0, sc0)

        compute(p0, a0, b0)
        issue_out(c0, b0, sb0, sc0)

        @pl.when(k < NPAIR - 1)
        def _i0():
            issue_loads(c0 + 2, p0, a0, gp0, ga0)

        drain_loads(c1, p1b, a1, gp1, ga1)

        @pl.when(k > 0)
        def _w1():
            drain_out(c1 - 2, b1, sb1, sc1)

        compute(p1b, a1, b1)
        issue_out(c1, b1, sb1, sc1)
        return 0

    lax.fori_loop(0, NPAIR, pair, 0)
    drain_out(NCHUNK - 2, b0, sb0, sc0)
    drain_out(NCHUNK - 1, b1, sb1, sc1)
    plsc.subcore_barrier()

    def cout(k, _):
        @pl.when(k < nchunks_own)
        def _do():
            r0 = row0 + k * ZR
            pltpu.sync_copy(cnt.at[pl.ds(r0, ZR), :],
                            cnt_hbm.at[cid, pl.ds(r0, ZR), :])
        return 0

    lax.fori_loop(0, 8, cout, 0)


def _tc_apply_body(p_ref, c_ref, xd_ref, wa_t_ref, we3_t_ref, ba_ref,
                   h_ref, p3_ref):
    s = p_ref[0] + p_ref[1]
    cnt = jnp.maximum(c_ref[0, :, :1] + c_ref[1, :, :1], 1.0)
    hn = s / cnt
    h_ref[:] = (jnp.dot(xd_ref[:], wa_t_ref[:D, :],
                        preferred_element_type=jnp.float32)
                + jnp.dot(hn, wa_t_ref[D:, :],
                          preferred_element_type=jnp.float32)
                + ba_ref[:])
    p3_ref[:] = jnp.dot(hn, we3_t_ref[:], preferred_element_type=jnp.float32)


def _tc_apply(partials, cnts, x_dst, wa_t, we3_t, ba):
    return pl.pallas_call(
        _tc_apply_body,
        out_shape=[
            jax.ShapeDtypeStruct((N, D), jnp.float32),
            jax.ShapeDtypeStruct((N, DE), jnp.float32),
        ],
    )(partials, cnts, x_dst, wa_t, we3_t, ba)


@functools.partial(
    pl.kernel,
    out_type=jax.ShapeDtypeStruct((E, DE), jnp.float32),
    mesh=mesh,
    scratch_types=[
        pltpu.VMEM((NCHUNK, C), jnp.int32),
        pltpu.VMEM((C, DE), jnp.float32),
        pltpu.VMEM((C, DE), jnp.float32),
        pltpu.VMEM((C, DE), jnp.float32),
        pltpu.VMEM((C, DE), jnp.float32),
        pltpu.VMEM((C, DE), jnp.float32),
        pltpu.VMEM((C, DE), jnp.float32),
        pltpu.SemaphoreType.DMA,
        pltpu.SemaphoreType.DMA,
        pltpu.SemaphoreType.DMA,
        pltpu.SemaphoreType.DMA,
        pltpu.SemaphoreType.DMA,
        pltpu.SemaphoreType.DMA,
    ],
    compiler_params=pltpu.CompilerParams(use_tc_tiling_on_sc=False),
)
def _sc_edge_out(b_hbm, p3_hbm, dst2_hbm, eh_hbm,
                 didx, p0, p1b, b0, b1, e0, e1,
                 gp0, gp1, gb0, gb1, ss0, ss1):
    cid = lax.axis_index("c")
    sid = lax.axis_index("s")
    wid = sid * NC + cid
    ebase = wid * EPW
    pltpu.sync_copy(dst2_hbm.at[pl.ds(wid * NCHUNK, NCHUNK), :], didx)

    def issue_loads(c, pb, bb, psem, bsem):
        pltpu.async_copy(p3_hbm.at[didx.at[c]], pb, psem)
        pltpu.async_copy(b_hbm.at[pl.ds(ebase + c * C, C), :], bb, bsem)

    def drain_loads(c, pb, bb, psem, bsem):
        pltpu.make_async_copy(p3_hbm.at[didx.at[c]], pb, psem).wait()
        pltpu.make_async_copy(b_hbm.at[pl.ds(ebase + c * C, C), :],
                              bb, bsem).wait()

    def compute(pb, bb, eb):
        def body(e, _):
            eb[e, :] = pb[e, :] + bb[e, :]
            return 0

        lax.fori_loop(0, C, body, 0)

    def issue_out(c, eb, sem):
        pltpu.async_copy(eb, eh_hbm.at[pl.ds(ebase + c * C, C), :], sem)

    def drain_out(c, eb, sem):
        pltpu.make_async_copy(eb, eh_hbm.at[pl.ds(ebase + c * C, C), :],
                              sem).wait()

    issue_loads(0, p0, b0, gp0, gb0)

    def pair(k, _):
        c0 = 2 * k
        c1 = 2 * k + 1
        issue_loads(c1, p1b, b1, gp1, gb1)
        drain_loads(c0, p0, b0, gp0, gb0)

        @pl.when(k > 0)
        def _w0():
            drain_out(c0 - 2, e0, ss0)

        compute(p0, b0, e0)
        issue_out(c0, e0, ss0)

        @pl.when(k < NPAIR - 1)
        def _i0():
            issue_loads(c0 + 2, p0, b0, gp0, gb0)

        drain_loads(c1, p1b, b1, gp1, gb1)

        @pl.when(k > 0)
        def _w1():
            drain_out(c1 - 2, e1, ss1)

        compute(p1b, b1, e1)
        issue_out(c1, e1, ss1)
        return 0

    lax.fori_loop(0, NPAIR, pair, 0)
    drain_out(NCHUNK - 2, e0, ss0)
    drain_out(NCHUNK - 1, e1, ss1)


def kernel(x_src, x_dst, edge_attr, W_msg, b_msg, W_apply, b_apply,
           W_edge, b_edge, edge_index):
    wn_t = jnp.concatenate([W_msg[:, :D], W_edge[:, :D]], axis=0).T
    we_t = jnp.concatenate([W_msg[:, D:], W_edge[:, D:D + DE]], axis=0).T
    be = jnp.concatenate([b_msg, b_edge]).reshape(1, D + DE)
    wa_t = W_apply.T
    we3_t = W_edge[:, D + DE:].T
    ba = b_apply.reshape(1, D)
    src2 = edge_index[0].reshape(E // C, C)
    dst2 = edge_index[1].reshape(E // C, C)
    src2s = edge_index[0].reshape(E // CS, CS)
    dst2s = edge_index[1].reshape(E // CS, CS)

    q1, p1 = _tc_node(x_src, wn_t)
    a, a2 = _tc_edge(edge_attr, we_t, be)
    partials = _sc_scatter(q1, a, src2s, dst2s)
    b, cnts = _sc_count_b(p1, a2, src2, dst2)
    h_out, p3 = _tc_apply(partials, cnts, x_dst, wa_t, we3_t, ba)
    eh = _sc_edge_out(b, p3, dst2)
    return (h_out, eh)


# speedup vs baseline: 3.4751x; 1.0118x over previous
import functools

import jax
import jax.numpy as jnp
from jax import lax
from jax.experimental import pallas as pl
from jax.experimental.pallas import tpu as pltpu
from jax.experimental.pallas import tpu_sc as plsc

N = 10000
E = 320000
D = 128
DE = 16

NC = 2
NS = 16
NW = NC * NS
EPW = E // NW
C = 100
NCHUNK = EPW // C
NPAIR = NCHUNK // 2
CS = 25
NCHUNKS = EPW // CS
NPAIRS = NCHUNKS // 2
OWN = 640
ZR = 80
ZB = 16
mesh = plsc.VectorSubcoreMesh(core_axis_name="c", subcore_axis_name="s")


def _tc_node_body(x_ref, w_ref, q1_ref, p1_ref):
    xw = jnp.dot(x_ref[:], w_ref[:], preferred_element_type=jnp.float32)
    q1_ref[:] = xw[:, :D]
    p1_ref[:] = xw[:, D:]


def _tc_node(x_src, wn_t):
    return pl.pallas_call(
        _tc_node_body,
        out_shape=[
            jax.ShapeDtypeStruct((N, D), jnp.float32),
            jax.ShapeDtypeStruct((N, DE), jnp.float32),
        ],
    )(x_src, wn_t)


def _tc_edge_body(attr_ref, w_ref, b_ref, a_ref, a2_ref):
    aw = jnp.dot(attr_ref[:], w_ref[:], preferred_element_type=jnp.float32)
    aw = aw + b_ref[:]
    a_ref[:] = aw[:, :D]
    a2_ref[:] = aw[:, D:]


def _tc_edge(edge_attr, we_t, bias):
    BE = 16000
    grid = E // BE
    return pl.pallas_call(
        _tc_edge_body,
        grid=(grid,),
        in_specs=[
            pl.BlockSpec((BE, DE), lambda i: (i, 0)),
            pl.BlockSpec((DE, D + DE), lambda i: (0, 0)),
            pl.BlockSpec((1, D + DE), lambda i: (0, 0)),
        ],
        out_specs=[
            pl.BlockSpec((BE, D), lambda i: (i, 0)),
            pl.BlockSpec((BE, DE), lambda i: (i, 0)),
        ],
        out_shape=[
            jax.ShapeDtypeStruct((E, D), jnp.float32),
            jax.ShapeDtypeStruct((E, DE), jnp.float32),
        ],
    )(edge_attr, we_t, bias)


@functools.partial(
    pl.kernel,
    out_type=jax.ShapeDtypeStruct((NC, N, D), jnp.float32),
    mesh=mesh,
    scratch_types=[
        pltpu.VMEM((NCHUNKS, CS), jnp.int32),
        pltpu.VMEM((NCHUNKS, CS), jnp.int32),
        pltpu.VMEM((CS, D), jnp.float32),
        pltpu.VMEM((CS, D), jnp.float32),
        pltpu.VMEM((CS, D), jnp.float32),
        pltpu.VMEM((CS, D), jnp.float32),
        pltpu.VMEM((CS, D), jnp.float32),
        pltpu.VMEM((CS, D), jnp.float32),
        pltpu.VMEM((ZB, D), jnp.float32),
        pltpu.VMEM_SHARED((N, D), jnp.float32),
        pltpu.SemaphoreType.DMA,
        pltpu.SemaphoreType.DMA,
        pltpu.SemaphoreType.DMA,
        pltpu.SemaphoreType.DMA,
        pltpu.SemaphoreType.DMA,
        pltpu.SemaphoreType.DMA,
    ],
    compiler_params=pltpu.CompilerParams(use_tc_tiling_on_sc=False),
)
def _sc_scatter(q1_hbm, a_hbm, ei_hbm, out_hbm,
                sidx, didx, q0, q1b, a0, a1, m0, m1, zbuf, acc,
                gq0, gq1, ga0, ga1, ss0, ss1):
    cid = lax.axis_index("c")
    sid = lax.axis_index("s")
    wid = sid * NC + cid
    row0 = sid * OWN
    nown = jnp.where(sid == NS - 1, N - (NS - 1) * OWN, OWN)
    nchunks_own = nown // ZR
    nzb_own = nown // ZB
    zv = jnp.zeros((16,), jnp.float32)

    def zrow(e, _):
        for j in range(D // 16):
            zbuf[e, pl.ds(j * 16, 16)] = zv
        return 0

    lax.fori_loop(0, ZB, zrow, 0)

    def zacc(k, _):
        @pl.when(k < nzb_own)
        def _do():
            pltpu.sync_copy(zbuf, acc.at[pl.ds(row0 + k * ZB, ZB), :])
        return 0

    lax.fori_loop(0, OWN // ZB, zacc, 0)

    ebase = wid * EPW
    pltpu.sync_copy(ei_hbm.at[0, pl.ds(wid * NCHUNKS, NCHUNKS), :], sidx)
    pltpu.sync_copy(ei_hbm.at[1, pl.ds(wid * NCHUNKS, NCHUNKS), :], didx)
    plsc.subcore_barrier()

    def issue_loads(c, qb, ab, qsem, asem):
        pltpu.async_copy(q1_hbm.at[sidx.at[c]], qb, qsem)
        pltpu.async_copy(a_hbm.at[pl.ds(ebase + c * CS, CS), :], ab, asem)

    def drain_loads(c, qb, ab, qsem, asem):
        pltpu.make_async_copy(q1_hbm.at[sidx.at[c]], qb, qsem).wait()
        pltpu.make_async_copy(a_hbm.at[pl.ds(ebase + c * CS, CS), :],
                              ab, asem).wait()

    def compute(qb, ab, mb):
        def body(e, _):
            for j in range(D // 16):
                sl = pl.ds(j * 16, 16)
                mb[e, sl] = jnp.maximum(qb[e, sl] + ab[e, sl], 0.0)
            return 0

        lax.fori_loop(0, CS, body, 0)

    def issue_scatter(c, mb, sem):
        pltpu.async_copy(mb, acc.at[didx.at[c]], sem, add=True)

    def drain_scatter(c, mb, sem):
        pltpu.make_async_copy(mb, acc.at[didx.at[c]], sem).wait()

    issue_loads(0, q0, a0, gq0, ga0)

    def pair(k, _):
        c0 = 2 * k
        c1 = 2 * k + 1
        issue_loads(c1, q1b, a1, gq1, ga1)
        drain_loads(c0, q0, a0, gq0, ga0)

        @pl.when(k > 0)
        def _w0():
            drain_scatter(c0 - 2, m0, ss0)

        compute(q0, a0, m0)
        issue_scatter(c0, m0, ss0)

        @pl.when(k < NPAIRS - 1)
        def _i0():
            issue_loads(c0 + 2, q0, a0, gq0, ga0)

        drain_loads(c1, q1b, a1, gq1, ga1)

        @pl.when(k > 0)
        def _w1():
            drain_scatter(c1 - 2, m1, ss1)

        compute(q1b, a1, m1)
        issue_scatter(c1, m1, ss1)
        return 0

    lax.fori_loop(0, NPAIRS, pair, 0)
    drain_scatter(NCHUNKS - 2, m0, ss0)
    drain_scatter(NCHUNKS - 1, m1, ss1)
    plsc.subcore_barrier()

    def cout(k, _):
        @pl.when(k < nchunks_own)
        def _do():
            r0 = row0 + k * ZR
            pltpu.sync_copy(acc.at[pl.ds(r0, ZR), :],
                            out_hbm.at[cid, pl.ds(r0, ZR), :])
        return 0

    lax.fori_loop(0, 8, cout, 0)


@functools.partial(
    pl.kernel,
    out_type=[
        jax.ShapeDtypeStruct((E, DE), jnp.float32),
        jax.ShapeDtypeStruct((NC, N, DE), jnp.float32),
    ],
    mesh=mesh,
    scratch_types=[
        pltpu.VMEM((NCHUNK, C), jnp.int32),
        pltpu.VMEM((NCHUNK, C), jnp.int32),
        pltpu.VMEM((C, DE), jnp.float32),
        pltpu.VMEM((C, DE), jnp.float32),
        pltpu.VMEM((C, DE), jnp.float32),
        pltpu.VMEM((C, DE), jnp.float32),
        pltpu.VMEM((C, DE), jnp.float32),
        pltpu.VMEM((C, DE), jnp.float32),
        pltpu.VMEM((C, DE), jnp.float32),
        pltpu.VMEM((ZR, DE), jnp.float32),
        pltpu.VMEM_SHARED((N, DE), jnp.float32),
        pltpu.SemaphoreType.DMA,
        pltpu.SemaphoreType.DMA,
        pltpu.SemaphoreType.DMA,
        pltpu.SemaphoreType.DMA,
        pltpu.SemaphoreType.DMA,
        pltpu.SemaphoreType.DMA,
        pltpu.SemaphoreType.DMA,
        pltpu.SemaphoreType.DMA,
    ],
    compiler_params=pltpu.CompilerParams(use_tc_tiling_on_sc=False),
)
def _sc_count_b(p1_hbm, a2_hbm, ei_hbm, b_hbm, cnt_hbm,
                sidx, didx, p0, p1b, a0, a1, b0, b1, onesr, zbuf, cnt,
                gp0, gp1, ga0, ga1, sb0, sb1, sc0, sc1):
    cid = lax.axis_index("c")
    sid = lax.axis_index("s")
    wid = sid * NC + cid
    row0 = sid * OWN
    nown = jnp.where(sid == NS - 1, N - (NS - 1) * OWN, OWN)
    nchunks_own = nown // ZR
    zv = jnp.zeros((16,), jnp.float32)
    onev = jnp.where(jnp.arange(16, dtype=jnp.int32) == 0,
                     jnp.float32(1.0), jnp.float32(0.0))

    def zrow(e, _):
        zbuf[e, :] = zv
        return 0

    lax.fori_loop(0, ZR, zrow, 0)

    def orow(e, _):
        onesr[e, :] = onev
        return 0

    lax.fori_loop(0, C, orow, 0)

    def zacc(k, _):
        @pl.when(k < nchunks_own)
        def _do():
            pltpu.sync_copy(zbuf, cnt.at[pl.ds(row0 + k * ZR, ZR), :])
        return 0

    lax.fori_loop(0, 8, zacc, 0)
    ebase = wid * EPW
    pltpu.sync_copy(ei_hbm.at[0, pl.ds(wid * NCHUNK, NCHUNK), :], sidx)
    pltpu.sync_copy(ei_hbm.at[1, pl.ds(wid * NCHUNK, NCHUNK), :], didx)
    plsc.subcore_barrier()

    def issue_loads(c, pb, ab, psem, asem):
        pltpu.async_copy(p1_hbm.at[sidx.at[c]], pb, psem)
        pltpu.async_copy(a2_hbm.at[pl.ds(ebase + c * C, C), :], ab, asem)

    def drain_loads(c, pb, ab, psem, asem):
        pltpu.make_async_copy(p1_hbm.at[sidx.at[c]], pb, psem).wait()
        pltpu.make_async_copy(a2_hbm.at[pl.ds(ebase + c * C, C), :],
                              ab, asem).wait()

    def compute(pb, ab, bb):
        def body(e, _):
            bb[e, :] = pb[e, :] + ab[e, :]
            return 0

        lax.fori_loop(0, C, body, 0)

    def issue_out(c, bb, bsem, csem):
        pltpu.async_copy(bb, b_hbm.at[pl.ds(ebase + c * C, C), :], bsem)
        pltpu.async_copy(onesr, cnt.at[didx.at[c]], csem, add=True)

    def drain_out(c, bb, bsem, csem):
        pltpu.make_async_copy(bb, b_hbm.at[pl.ds(ebase + c * C, C), :],
                              bsem).wait()
        pltpu.make_async_copy(onesr, cnt.at[didx.at[c]], csem).wait()

    issue_loads(0, p0, a0, gp0, ga0)

    def pair(k, _):
        c0 = 2 * k
        c1 = 2 * k + 1
        issue_loads(c1, p1b, a1, gp1, ga1)
        drain_loads(c0, p0, a0, gp0, ga0)

        @pl.when(k > 0)
        def _w0():
            drain_out(c0 - 2, b0, sb0, sc0)

        compute(p0, a0, b0)
        issue_out(c0, b0, sb0, sc0)

        @pl.when(k < NPAIR - 1)
        def _i0():
            issue_loads(c0 + 2, p0, a0, gp0, ga0)

        drain_loads(c1, p1b, a1, gp1, ga1)

        @pl.when(k > 0)
        def _w1():
            drain_out(c1 - 2, b1, sb1, sc1)

        compute(p1b, a1, b1)
        issue_out(c1, b1, sb1, sc1)
        return 0

    lax.fori_loop(0, NPAIR, pair, 0)
    drain_out(NCHUNK - 2, b0, sb0, sc0)
    drain_out(NCHUNK - 1, b1, sb1, sc1)
    plsc.subcore_barrier()

    def cout(k, _):
        @pl.when(k < nchunks_own)
        def _do():
            r0 = row0 + k * ZR
            pltpu.sync_copy(cnt.at[pl.ds(r0, ZR), :],
                            cnt_hbm.at[cid, pl.ds(r0, ZR), :])
        return 0

    lax.fori_loop(0, 8, cout, 0)


def _tc_apply_body(p_ref, c_ref, xd_ref, wa_t_ref, we3_t_ref, ba_ref,
                   h_ref, p3_ref):
    s = p_ref[0] + p_ref[1]
    cnt = jnp.maximum(c_ref[0, :, :1] + c_ref[1, :, :1], 1.0)
    hn = s / cnt
    h_ref[:] = (jnp.dot(xd_ref[:], wa_t_ref[:D, :],
                        preferred_element_type=jnp.float32)
                + jnp.dot(hn, wa_t_ref[D:, :],
                          preferred_element_type=jnp.float32)
                + ba_ref[:])
    p3_ref[:] = jnp.dot(hn, we3_t_ref[:], preferred_element_type=jnp.float32)


def _tc_apply(partials, cnts, x_dst, wa_t, we3_t, ba):
    return pl.pallas_call(
        _tc_apply_body,
        out_shape=[
            jax.ShapeDtypeStruct((N, D), jnp.float32),
            jax.ShapeDtypeStruct((N, DE), jnp.float32),
        ],
    )(partials, cnts, x_dst, wa_t, we3_t, ba)


@functools.partial(
    pl.kernel,
    out_type=jax.ShapeDtypeStruct((E, DE), jnp.float32),
    mesh=mesh,
    scratch_types=[
        pltpu.VMEM((NCHUNK, C), jnp.int32),
        pltpu.VMEM((C, DE), jnp.float32),
        pltpu.VMEM((C, DE), jnp.float32),
        pltpu.VMEM((C, DE), jnp.float32),
        pltpu.VMEM((C, DE), jnp.float32),
        pltpu.VMEM((C, DE), jnp.float32),
        pltpu.VMEM((C, DE), jnp.float32),
        pltpu.SemaphoreType.DMA,
        pltpu.SemaphoreType.DMA,
        pltpu.SemaphoreType.DMA,
        pltpu.SemaphoreType.DMA,
        pltpu.SemaphoreType.DMA,
        pltpu.SemaphoreType.DMA,
    ],
    compiler_params=pltpu.CompilerParams(use_tc_tiling_on_sc=False),
)
def _sc_edge_out(b_hbm, p3_hbm, ei_hbm, eh_hbm,
                 didx, p0, p1b, b0, b1, e0, e1,
                 gp0, gp1, gb0, gb1, ss0, ss1):
    cid = lax.axis_index("c")
    sid = lax.axis_index("s")
    wid = sid * NC + cid
    ebase = wid * EPW
    pltpu.sync_copy(ei_hbm.at[1, pl.ds(wid * NCHUNK, NCHUNK), :], didx)

    def issue_loads(c, pb, bb, psem, bsem):
        pltpu.async_copy(p3_hbm.at[didx.at[c]], pb, psem)
        pltpu.async_copy(b_hbm.at[pl.ds(ebase + c * C, C), :], bb, bsem)

    def drain_loads(c, pb, bb, psem, bsem):
        pltpu.make_async_copy(p3_hbm.at[didx.at[c]], pb, psem).wait()
        pltpu.make_async_copy(b_hbm.at[pl.ds(ebase + c * C, C), :],
                              bb, bsem).wait()

    def compute(pb, bb, eb):
        def body(e, _):
            eb[e, :] = pb[e, :] + bb[e, :]
            return 0

        lax.fori_loop(0, C, body, 0)

    def issue_out(c, eb, sem):
        pltpu.async_copy(eb, eh_hbm.at[pl.ds(ebase + c * C, C), :], sem)

    def drain_out(c, eb, sem):
        pltpu.make_async_copy(eb, eh_hbm.at[pl.ds(ebase + c * C, C), :],
                              sem).wait()

    issue_loads(0, p0, b0, gp0, gb0)

    def pair(k, _):
        c0 = 2 * k
        c1 = 2 * k + 1
        issue_loads(c1, p1b, b1, gp1, gb1)
        drain_loads(c0, p0, b0, gp0, gb0)

        @pl.when(k > 0)
        def _w0():
            drain_out(c0 - 2, e0, ss0)

        compute(p0, b0, e0)
        issue_out(c0, e0, ss0)

        @pl.when(k < NPAIR - 1)
        def _i0():
            issue_loads(c0 + 2, p0, b0, gp0, gb0)

        drain_loads(c1, p1b, b1, gp1, gb1)

        @pl.when(k > 0)
        def _w1():
            drain_out(c1 - 2, e1, ss1)

        compute(p1b, b1, e1)
        issue_out(c1, e1, ss1)
        return 0

    lax.fori_loop(0, NPAIR, pair, 0)
    drain_out(NCHUNK - 2, e0, ss0)
    drain_out(NCHUNK - 1, e1, ss1)


def kernel(x_src, x_dst, edge_attr, W_msg, b_msg, W_apply, b_apply,
           W_edge, b_edge, edge_index):
    wn_t = jnp.concatenate([W_msg[:, :D], W_edge[:, :D]], axis=0).T
    we_t = jnp.concatenate([W_msg[:, D:], W_edge[:, D:D + DE]], axis=0).T
    be = jnp.concatenate([b_msg, b_edge]).reshape(1, D + DE)
    wa_t = W_apply.T
    we3_t = W_edge[:, D + DE:].T
    ba = b_apply.reshape(1, D)
    ei3s = edge_index.reshape(2, E // CS, CS)
    ei3 = edge_index.reshape(2, E // C, C)

    q1, p1 = _tc_node(x_src, wn_t)
    a, a2 = _tc_edge(edge_attr, we_t, be)
    partials = _sc_scatter(q1, a, ei3s)
    b, cnts = _sc_count_b(p1, a2, ei3)
    h_out, p3 = _tc_apply(partials, cnts, x_dst, wa_t, we3_t, ba)
    eh = _sc_edge_out(b, p3, ei3)
    return (h_out, eh)
